# baseline (speedup 1.0000x reference)
import functools

import jax
import jax.numpy as jnp
from jax import lax
from jax.experimental import pallas as pl
from jax.experimental.pallas import tpu as pltpu
from jax.experimental.pallas import tpu_sc as plsc

N = 10000
D = 128
E = 320000

CHUNK = 112
NWORKERS = 32
CPT = 90
EPT = CPT * CHUNK
E_PAD = NWORKERS * EPT
TRIPLES = CPT // 3
ACC_ROWS = 10112
ROWS_PER_TILE = ACC_ROWS // 16


@functools.partial(
    pl.kernel,
    mesh=plsc.VectorSubcoreMesh(core_axis_name="c", subcore_axis_name="s"),
    out_type=jax.ShapeDtypeStruct((2, ACC_ROWS, D), jnp.float32),
    scratch_types=[
        pltpu.VMEM((2, CHUNK), jnp.int32),
        pltpu.VMEM((2, CHUNK), jnp.int32),
        pltpu.VMEM((2, CHUNK), jnp.int32),
        pltpu.VMEM((CHUNK,), jnp.float32),
        pltpu.VMEM((CHUNK,), jnp.float32),
        pltpu.VMEM((CHUNK,), jnp.float32),
        pltpu.VMEM((CHUNK, D), jnp.float32),
        pltpu.VMEM((CHUNK, D), jnp.float32),
        pltpu.VMEM((CHUNK, D), jnp.float32),
        pltpu.VMEM_SHARED((ACC_ROWS, D), jnp.float32),
        pltpu.SemaphoreType.DMA,
        pltpu.SemaphoreType.DMA,
        pltpu.SemaphoreType.DMA,
        pltpu.SemaphoreType.DMA,
        pltpu.SemaphoreType.DMA,
        pltpu.SemaphoreType.DMA,
    ],
)
def _sc_aggregate(idx_hbm, w_hbm, ego_hbm, zeros_hbm, out_hbm,
                  m0, m1, m2, w0, w1, w2, r0, r1, r2, acc,
                  g0, g1, g2, s0, s1, s2):
    c = lax.axis_index("c")
    s = lax.axis_index("s")
    wid = s * 2 + c
    rows = (r0, r1, r2)
    metas = (m0, m1, m2)
    ws = (w0, w1, w2)
    gsems = (g0, g1, g2)
    ssems = (s0, s1, s2)

    row0 = s * ROWS_PER_TILE
    pltpu.sync_copy(zeros_hbm, acc.at[pl.ds(row0, ROWS_PER_TILE)])

    def stage_meta(b, k):
        chunk_id = wid * CPT + k
        pltpu.sync_copy(idx_hbm.at[chunk_id], metas[b])
        pltpu.sync_copy(w_hbm.at[pl.ds(chunk_id * CHUNK, CHUNK)], ws[b])

    def fire_g(b):
        pltpu.make_async_copy(ego_hbm.at[metas[b].at[0]], rows[b],
                              gsems[b]).start()

    def wait_g(b):
        pltpu.make_async_copy(ego_hbm.at[metas[b].at[0]], rows[b],
                              gsems[b]).wait()

    def fire_s(b):
        pltpu.make_async_copy(rows[b], acc.at[metas[b].at[1]],
                              ssems[b]).start(add=True)

    def wait_s(b):
        pltpu.make_async_copy(rows[b], acc.at[metas[b].at[1]],
                              ssems[b]).wait()

    def scale(b):
        def group_body(g, carry):
            w16 = ws[b][pl.ds(g * 16, 16)]
            for lane in range(16):
                e = g * 16 + lane
                w = jnp.full((16,), w16[lane])
                for j in range(D // 16):
                    sl = pl.ds(j * 16, 16)
                    rows[b][e, sl] = rows[b][e, sl] * w
            return carry

        lax.fori_loop(0, CHUNK // 16, group_body, 0)

    stage_meta(0, 0)
    fire_g(0)
    plsc.subcore_barrier()

    def triple_body(t, carry):
        for u in range(3):
            j = t * 3 + u
            nb = (u + 1) % 3
            if u == 2:
                wait_s(nb)

                @pl.when(t < TRIPLES - 1)
                def _():
                    stage_meta(nb, j + 1)
                    fire_g(nb)
            else:

                @pl.when(t >= 1)
                def _():
                    wait_s(nb)

                stage_meta(nb, j + 1)
                fire_g(nb)

            wait_g(u)
            scale(u)
            fire_s(u)
        return carry

    lax.fori_loop(0, TRIPLES, triple_body, 0)
    wait_s(1)
    wait_s(2)

    plsc.subcore_barrier()
    pltpu.sync_copy(acc.at[pl.ds(row0, ROWS_PER_TILE)],
                    out_hbm.at[c, pl.ds(row0, ROWS_PER_TILE)])


ROWS_BLK = 1000


def _tc_dense_body(ego_ref, p_ref, w1_ref, b1_ref, w2_ref, b2_ref, out_ref):
    side = p_ref[0] + p_ref[1]
    ego = ego_ref[...]
    dn = (((1,), (1,)), ((), ()))
    y1 = lax.dot_general(ego + side, w1_ref[...], dn,
                         preferred_element_type=jnp.float32) + b1_ref[...]
    y1 = jnp.where(y1 >= 0, y1, 0.01 * y1)
    y2 = lax.dot_general(ego * side, w2_ref[...], dn,
                         preferred_element_type=jnp.float32) + b2_ref[...]
    y2 = jnp.where(y2 >= 0, y2, 0.01 * y2)
    out_ref[...] = y1 + y2


_tc_dense = pl.pallas_call(
    _tc_dense_body,
    grid=(N // ROWS_BLK,),
    in_specs=[
        pl.BlockSpec((ROWS_BLK, D), lambda i: (i, 0)),
        pl.BlockSpec((2, ROWS_BLK, D), lambda i: (0, i, 0)),
        pl.BlockSpec((D, D), lambda i: (0, 0)),
        pl.BlockSpec((1, D), lambda i: (0, 0)),
        pl.BlockSpec((D, D), lambda i: (0, 0)),
        pl.BlockSpec((1, D), lambda i: (0, 0)),
    ],
    out_specs=pl.BlockSpec((ROWS_BLK, D), lambda i: (i, 0)),
    out_shape=jax.ShapeDtypeStruct((N, D), jnp.float32),
)


def kernel(ego_embeddings, edge_index, edge_weight, W1, b1, W2, b2):
    src = edge_index[0].astype(jnp.int32)
    dst = edge_index[1].astype(jnp.int32)
    pad = E_PAD - E
    srcp = jnp.concatenate([src, jnp.zeros((pad,), jnp.int32)])
    dstp = jnp.concatenate([dst, jnp.zeros((pad,), jnp.int32)])
    wp = jnp.concatenate([edge_weight, jnp.zeros((pad,), jnp.float32)])
    idx = (jnp.stack([srcp, dstp])
           .reshape(2, NWORKERS * CPT, CHUNK)
           .transpose(1, 0, 2))
    zeros = jnp.zeros((ROWS_PER_TILE, D), jnp.float32)
    partials = _sc_aggregate(idx, wp, ego_embeddings, zeros)
    return _tc_dense(ego_embeddings, partials, W1, b1.reshape(1, D),
                     W2, b2.reshape(1, D))


# speedup vs baseline: 1.0779x; 1.0779x over previous
import functools

import jax
import jax.numpy as jnp
from jax import lax
from jax.experimental import pallas as pl
from jax.experimental.pallas import tpu as pltpu
from jax.experimental.pallas import tpu_sc as plsc

N = 10000
D = 128
E = 320000

CHUNK = 112
NWORKERS = 32
CPT = 90
EPT = CPT * CHUNK
E_PAD = NWORKERS * EPT
TRIPLES = CPT // 3
ACC_ROWS = 10112
ROWS_PER_TILE = ACC_ROWS // 16


@functools.partial(
    pl.kernel,
    mesh=plsc.VectorSubcoreMesh(core_axis_name="c", subcore_axis_name="s"),
    out_type=jax.ShapeDtypeStruct((2, ACC_ROWS, D), jnp.float32),
    scratch_types=[
        pltpu.VMEM((2, CHUNK), jnp.int32),
        pltpu.VMEM((2, CHUNK), jnp.int32),
        pltpu.VMEM((2, CHUNK), jnp.int32),
        pltpu.VMEM((CHUNK,), jnp.float32),
        pltpu.VMEM((CHUNK,), jnp.float32),
        pltpu.VMEM((CHUNK,), jnp.float32),
        pltpu.VMEM((CHUNK, D), jnp.float32),
        pltpu.VMEM((CHUNK, D), jnp.float32),
        pltpu.VMEM((CHUNK, D), jnp.float32),
        pltpu.VMEM_SHARED((ACC_ROWS, D), jnp.float32),
        pltpu.SemaphoreType.DMA,
        pltpu.SemaphoreType.DMA,
        pltpu.SemaphoreType.DMA,
        pltpu.SemaphoreType.DMA,
        pltpu.SemaphoreType.DMA,
        pltpu.SemaphoreType.DMA,
    ],
)
def _sc_aggregate(idx_hbm, w_hbm, ego_hbm, zeros_hbm, out_hbm,
                  m0, m1, m2, w0, w1, w2, r0, r1, r2, acc,
                  g0, g1, g2, ms0, ms1, ms2):
    c = lax.axis_index("c")
    s = lax.axis_index("s")
    wid = s * 2 + c
    rows = (r0, r1, r2)
    metas = (m0, m1, m2)
    ws = (w0, w1, w2)
    gsems = (g0, g1, g2)
    msems = (ms0, ms1, ms2)

    row0 = s * ROWS_PER_TILE
    pltpu.sync_copy(zeros_hbm, acc.at[pl.ds(row0, ROWS_PER_TILE)])

    def fire_meta(b, k):
        chunk_id = wid * CPT + k
        pltpu.make_async_copy(idx_hbm.at[chunk_id], metas[b],
                              msems[b]).start()
        pltpu.make_async_copy(w_hbm.at[pl.ds(chunk_id * CHUNK, CHUNK)],
                              ws[b], msems[b]).start()

    def wait_meta(b):
        pltpu.make_async_copy(idx_hbm.at[0], metas[b], msems[b]).wait()
        pltpu.make_async_copy(w_hbm.at[pl.ds(0, CHUNK)], ws[b],
                              msems[b]).wait()

    def fire_g(b):
        pltpu.make_async_copy(ego_hbm.at[metas[b].at[0]], rows[b],
                              gsems[b]).start()

    def wait_g(b):
        pltpu.make_async_copy(ego_hbm.at[metas[b].at[0]], rows[b],
                              gsems[b]).wait()

    def scatter(b):
        pltpu.sync_copy(rows[b], acc.at[metas[b].at[1]], add=True)

    def scale(b):
        def group_body(g, carry):
            w16 = ws[b][pl.ds(g * 16, 16)]
            for lane in range(16):
                e = g * 16 + lane
                w = jnp.full((16,), w16[lane])
                for j in range(D // 16):
                    sl = pl.ds(j * 16, 16)
                    rows[b][e, sl] = rows[b][e, sl] * w
            return carry

        lax.fori_loop(0, CHUNK // 16, group_body, 0)

    fire_meta(0, 0)
    wait_meta(0)
    fire_g(0)
    fire_meta(1, 1)
    plsc.subcore_barrier()

    def triple_body(t, carry):
        last = TRIPLES - 1
        for u in range(3):
            j = t * 3 + u
            nb = (u + 1) % 3
            nn = (u + 2) % 3
            if u == 0:
                wait_meta(nb)
                fire_g(nb)
                fire_meta(nn, j + 2)
            elif u == 1:
                wait_meta(nb)
                fire_g(nb)

                @pl.when(t < last)
                def _():
                    fire_meta(nn, j + 2)

            else:
                @pl.when(t < last)
                def _():
                    wait_meta(nb)
                    fire_g(nb)
                    fire_meta(nn, j + 2)

            wait_g(u)
            scale(u)
            scatter(u)
        return carry

    lax.fori_loop(0, TRIPLES, triple_body, 0)
    plsc.subcore_barrier()
    pltpu.sync_copy(acc.at[pl.ds(row0, ROWS_PER_TILE)],
                    out_hbm.at[c, pl.ds(row0, ROWS_PER_TILE)])


ROWS_BLK = 1000


def _tc_dense_body(ego_ref, p_ref, w1_ref, b1_ref, w2_ref, b2_ref, out_ref):
    side = p_ref[0] + p_ref[1]
    ego = ego_ref[...]
    dn = (((1,), (1,)), ((), ()))
    y1 = lax.dot_general(ego + side, w1_ref[...], dn,
                         preferred_element_type=jnp.float32) + b1_ref[...]
    y1 = jnp.where(y1 >= 0, y1, 0.01 * y1)
    y2 = lax.dot_general(ego * side, w2_ref[...], dn,
                         preferred_element_type=jnp.float32) + b2_ref[...]
    y2 = jnp.where(y2 >= 0, y2, 0.01 * y2)
    out_ref[...] = y1 + y2


_tc_dense = pl.pallas_call(
    _tc_dense_body,
    grid=(N // ROWS_BLK,),
    in_specs=[
        pl.BlockSpec((ROWS_BLK, D), lambda i: (i, 0)),
        pl.BlockSpec((2, ROWS_BLK, D), lambda i: (0, i, 0)),
        pl.BlockSpec((D, D), lambda i: (0, 0)),
        pl.BlockSpec((1, D), lambda i: (0, 0)),
        pl.BlockSpec((D, D), lambda i: (0, 0)),
        pl.BlockSpec((1, D), lambda i: (0, 0)),
    ],
    out_specs=pl.BlockSpec((ROWS_BLK, D), lambda i: (i, 0)),
    out_shape=jax.ShapeDtypeStruct((N, D), jnp.float32),
)


def kernel(ego_embeddings, edge_index, edge_weight, W1, b1, W2, b2):
    src = edge_index[0].astype(jnp.int32)
    dst = edge_index[1].astype(jnp.int32)
    pad = E_PAD - E
    srcp = jnp.concatenate([src, jnp.zeros((pad,), jnp.int32)])
    dstp = jnp.concatenate([dst, jnp.zeros((pad,), jnp.int32)])
    wp = jnp.concatenate([edge_weight, jnp.zeros((pad,), jnp.float32)])
    idx = (jnp.stack([srcp, dstp])
           .reshape(2, NWORKERS * CPT, CHUNK)
           .transpose(1, 0, 2))
    zeros = jnp.zeros((ROWS_PER_TILE, D), jnp.float32)
    partials = _sc_aggregate(idx, wp, ego_embeddings, zeros)
    return _tc_dense(ego_embeddings, partials, W1, b1.reshape(1, D),
                     W2, b2.reshape(1, D))


# speedup vs baseline: 1.4922x; 1.3843x over previous
import functools

import jax
import jax.numpy as jnp
from jax import lax
from jax.experimental import pallas as pl
from jax.experimental.pallas import tpu as pltpu
from jax.experimental.pallas import tpu_sc as plsc

N = 10000
D = 128
E = 320000

CHUNK = 128
NCHUNKS = E // CHUNK
NWORKERS = 32
CHUNKS_PER_TILE = (NCHUNKS + NWORKERS - 1) // NWORKERS
ACC_ROWS = 10240
ROWS_PER_TILE = ACC_ROWS // 16


PAIRS = (CHUNKS_PER_TILE + 1) // 2


@functools.partial(
    pl.kernel,
    mesh=plsc.VectorSubcoreMesh(core_axis_name="c", subcore_axis_name="s"),
    out_type=jax.ShapeDtypeStruct((2, ACC_ROWS, D), jnp.float32),
    scratch_types=[
        pltpu.VMEM((2, CHUNK), jnp.int32),
        pltpu.VMEM((2, CHUNK), jnp.int32),
        pltpu.VMEM((CHUNK,), jnp.float32),
        pltpu.VMEM((CHUNK,), jnp.float32),
        pltpu.VMEM((CHUNK, D), jnp.float32),
        pltpu.VMEM((CHUNK, D), jnp.float32),
        pltpu.VMEM_SHARED((ACC_ROWS, D), jnp.float32),
        pltpu.SemaphoreType.DMA,
        pltpu.SemaphoreType.DMA,
        pltpu.SemaphoreType.DMA,
    ],
)
def _sc_aggregate(meta_hbm, w_hbm, ego_hbm, zeros_hbm, out_hbm,
                  meta0, meta1, w0, w1, rows0, rows1, acc, gsem0, gsem1,
                  msem):
    c = lax.axis_index("c")
    s = lax.axis_index("s")
    wid = s * 2 + c
    metas = (meta0, meta1)
    ws = (w0, w1)
    rows = (rows0, rows1)
    gsems = (gsem0, gsem1)

    row0 = s * ROWS_PER_TILE
    pltpu.sync_copy(zeros_hbm, acc.at[pl.ds(row0, ROWS_PER_TILE)])
    plsc.subcore_barrier()

    def fire(b, chunk_id):
        pltpu.make_async_copy(meta_hbm.at[chunk_id], metas[b], msem).start()
        pltpu.make_async_copy(w_hbm.at[pl.ds(chunk_id * CHUNK, CHUNK)],
                              ws[b], msem).start()
        pltpu.make_async_copy(meta_hbm.at[0], metas[b], msem).wait()
        pltpu.make_async_copy(w_hbm.at[pl.ds(0, CHUNK)], ws[b], msem).wait()
        pltpu.make_async_copy(ego_hbm.at[metas[b].at[0]], rows[b],
                              gsems[b]).start()

    def process(b):
        pltpu.make_async_copy(ego_hbm.at[metas[b].at[0]], rows[b],
                              gsems[b]).wait()

        def group_body(g, carry2):
            w16 = ws[b][pl.ds(g * 16, 16)]
            for lane in range(16):
                e = g * 16 + lane
                w = jnp.full((16,), w16[lane])
                for j in range(D // 16):
                    sl = pl.ds(j * 16, 16)
                    rows[b][e, sl] = rows[b][e, sl] * w
            return carry2

        lax.fori_loop(0, CHUNK // 16, group_body, 0)
        pltpu.sync_copy(rows[b], acc.at[metas[b].at[1]], add=True)

    fire(0, wid)

    def pair_body(p, carry):
        for b in range(2):
            k = p * 2 + b
            cur = k * NWORKERS + wid

            @pl.when(cur < NCHUNKS)
            def _():
                nxt = cur + NWORKERS

                @pl.when(nxt < NCHUNKS)
                def _():
                    fire(1 - b, nxt)

                process(b)

        return carry

    lax.fori_loop(0, PAIRS, pair_body, 0)
    plsc.subcore_barrier()
    pltpu.sync_copy(acc.at[pl.ds(row0, ROWS_PER_TILE)],
                    out_hbm.at[c, pl.ds(row0, ROWS_PER_TILE)])


ROWS_BLK = 1000


def _tc_dense_body(ego_ref, p_ref, w1_ref, b1_ref, w2_ref, b2_ref, out_ref):
    side = p_ref[0] + p_ref[1]
    ego = ego_ref[...]
    dn = (((1,), (1,)), ((), ()))
    y1 = lax.dot_general(ego + side, w1_ref[...], dn,
                         preferred_element_type=jnp.float32) + b1_ref[...]
    y1 = jnp.where(y1 >= 0, y1, 0.01 * y1)
    y2 = lax.dot_general(ego * side, w2_ref[...], dn,
                         preferred_element_type=jnp.float32) + b2_ref[...]
    y2 = jnp.where(y2 >= 0, y2, 0.01 * y2)
    out_ref[...] = y1 + y2


_tc_dense = pl.pallas_call(
    _tc_dense_body,
    grid=(N // ROWS_BLK,),
    in_specs=[
        pl.BlockSpec((ROWS_BLK, D), lambda i: (i, 0)),
        pl.BlockSpec((2, ROWS_BLK, D), lambda i: (0, i, 0)),
        pl.BlockSpec((D, D), lambda i: (0, 0)),
        pl.BlockSpec((1, D), lambda i: (0, 0)),
        pl.BlockSpec((D, D), lambda i: (0, 0)),
        pl.BlockSpec((1, D), lambda i: (0, 0)),
    ],
    out_specs=pl.BlockSpec((ROWS_BLK, D), lambda i: (i, 0)),
    out_shape=jax.ShapeDtypeStruct((N, D), jnp.float32),
)


def kernel(ego_embeddings, edge_index, edge_weight, W1, b1, W2, b2):
    src = edge_index[0].astype(jnp.int32)
    dst = edge_index[1].astype(jnp.int32)
    meta = (jnp.stack([src, dst])
            .reshape(2, NCHUNKS, CHUNK)
            .transpose(1, 0, 2))
    zeros = jnp.zeros((ROWS_PER_TILE, D), jnp.float32)
    partials = _sc_aggregate(meta, edge_weight, ego_embeddings, zeros)
    return _tc_dense(ego_embeddings, partials, W1, b1.reshape(1, D),
                     W2, b2.reshape(1, D))


# speedup vs baseline: 1.7512x; 1.1736x over previous
import functools

import jax
import jax.numpy as jnp
from jax import lax
from jax.experimental import pallas as pl
from jax.experimental.pallas import tpu as pltpu
from jax.experimental.pallas import tpu_sc as plsc

N = 10000
D = 128
E = 320000

CHUNK = 128
NCHUNKS = E // CHUNK
NWORKERS = 32
CHUNKS_PER_TILE = (NCHUNKS + NWORKERS - 1) // NWORKERS
ACC_ROWS = 10240
ROWS_PER_TILE = ACC_ROWS // 16


PAIRS = (CHUNKS_PER_TILE + 1) // 2


@functools.partial(
    pl.kernel,
    mesh=plsc.VectorSubcoreMesh(core_axis_name="c", subcore_axis_name="s"),
    out_type=jax.ShapeDtypeStruct((2, ACC_ROWS, D), jnp.float32),
    scratch_types=[
        pltpu.VMEM((2, CHUNK), jnp.int32),
        pltpu.VMEM((2, CHUNK), jnp.int32),
        pltpu.VMEM((2, CHUNK), jnp.int32),
        pltpu.VMEM((CHUNK,), jnp.float32),
        pltpu.VMEM((CHUNK,), jnp.float32),
        pltpu.VMEM((CHUNK,), jnp.float32),
        pltpu.VMEM((CHUNK, D), jnp.float32),
        pltpu.VMEM((CHUNK, D), jnp.float32),
        pltpu.VMEM_SHARED((ACC_ROWS, D), jnp.float32),
        pltpu.SemaphoreType.DMA,
        pltpu.SemaphoreType.DMA,
        pltpu.SemaphoreType.DMA,
        pltpu.SemaphoreType.DMA,
        pltpu.SemaphoreType.DMA,
    ],
)
def _sc_aggregate(meta_hbm, w_hbm, ego_hbm, zeros_hbm, out_hbm,
                  meta0, meta1, meta2, w0, w1, w2, rows0, rows1, acc,
                  gsem0, gsem1, msem0, msem1, msem2):
    c = lax.axis_index("c")
    s = lax.axis_index("s")
    wid = s * 2 + c
    metas = (meta0, meta1, meta2)
    ws = (w0, w1, w2)
    rows = (rows0, rows1)
    gsems = (gsem0, gsem1)
    msems = (msem0, msem1, msem2)

    row0 = s * ROWS_PER_TILE
    pltpu.sync_copy(zeros_hbm, acc.at[pl.ds(row0, ROWS_PER_TILE)])
    plsc.subcore_barrier()

    def fetch_meta(m, chunk_id):
        pltpu.make_async_copy(meta_hbm.at[chunk_id], metas[m],
                              msems[m]).start()
        pltpu.make_async_copy(w_hbm.at[pl.ds(chunk_id * CHUNK, CHUNK)],
                              ws[m], msems[m]).start()

    def wait_meta(m):
        pltpu.make_async_copy(meta_hbm.at[0], metas[m], msems[m]).wait()
        pltpu.make_async_copy(w_hbm.at[pl.ds(0, CHUNK)], ws[m],
                              msems[m]).wait()

    def fire_gather(b, m):
        pltpu.make_async_copy(ego_hbm.at[metas[m].at[0]], rows[b],
                              gsems[b]).start()

    def process(b, m):
        pltpu.make_async_copy(ego_hbm.at[metas[0].at[0]], rows[b],
                              gsems[b]).wait()

        def group_body(g, carry2):
            w16 = ws[m][pl.ds(g * 16, 16)]
            for lane in range(16):
                e = g * 16 + lane
                w = jnp.full((16,), w16[lane])
                for j in range(D // 16):
                    sl = pl.ds(j * 16, 16)
                    rows[b][e, sl] = rows[b][e, sl] * w
            return carry2

        lax.fori_loop(0, CHUNK // 16, group_body, 0)
        pltpu.sync_copy(rows[b], acc.at[metas[m].at[1]], add=True)

    fetch_meta(0, wid)
    wait_meta(0)
    fire_gather(0, 0)
    fetch_meta(1, NWORKERS + wid)

    def block_body(p, carry):
        for u in range(6):
            k = p * 6 + u
            b = u % 2
            m1 = (u + 1) % 3
            m2 = (u + 2) % 3
            cur = k * NWORKERS + wid

            @pl.when(cur < NCHUNKS)
            def _():
                @pl.when(cur + NWORKERS < NCHUNKS)
                def _():
                    wait_meta(m1)
                    fire_gather(1 - b, m1)

                @pl.when(cur + 2 * NWORKERS < NCHUNKS)
                def _():
                    fetch_meta(m2, cur + 2 * NWORKERS)

                process(b, u % 3)

        return carry

    lax.fori_loop(0, (CHUNKS_PER_TILE + 5) // 6, block_body, 0)
    plsc.subcore_barrier()
    pltpu.sync_copy(acc.at[pl.ds(row0, ROWS_PER_TILE)],
                    out_hbm.at[c, pl.ds(row0, ROWS_PER_TILE)])


ROWS_BLK = 1000


def _tc_dense_body(ego_ref, p_ref, w1_ref, b1_ref, w2_ref, b2_ref, out_ref):
    side = p_ref[0] + p_ref[1]
    ego = ego_ref[...]
    dn = (((1,), (1,)), ((), ()))
    y1 = lax.dot_general(ego + side, w1_ref[...], dn,
                         preferred_element_type=jnp.float32) + b1_ref[...]
    y1 = jnp.where(y1 >= 0, y1, 0.01 * y1)
    y2 = lax.dot_general(ego * side, w2_ref[...], dn,
                         preferred_element_type=jnp.float32) + b2_ref[...]
    y2 = jnp.where(y2 >= 0, y2, 0.01 * y2)
    out_ref[...] = y1 + y2


_tc_dense = pl.pallas_call(
    _tc_dense_body,
    grid=(N // ROWS_BLK,),
    in_specs=[
        pl.BlockSpec((ROWS_BLK, D), lambda i: (i, 0)),
        pl.BlockSpec((2, ROWS_BLK, D), lambda i: (0, i, 0)),
        pl.BlockSpec((D, D), lambda i: (0, 0)),
        pl.BlockSpec((1, D), lambda i: (0, 0)),
        pl.BlockSpec((D, D), lambda i: (0, 0)),
        pl.BlockSpec((1, D), lambda i: (0, 0)),
    ],
    out_specs=pl.BlockSpec((ROWS_BLK, D), lambda i: (i, 0)),
    out_shape=jax.ShapeDtypeStruct((N, D), jnp.float32),
)


def kernel(ego_embeddings, edge_index, edge_weight, W1, b1, W2, b2):
    src = edge_index[0].astype(jnp.int32)
    dst = edge_index[1].astype(jnp.int32)
    meta = (jnp.stack([src, dst])
            .reshape(2, NCHUNKS, CHUNK)
            .transpose(1, 0, 2))
    zeros = jnp.zeros((ROWS_PER_TILE, D), jnp.float32)
    partials = _sc_aggregate(meta, edge_weight, ego_embeddings, zeros)
    return _tc_dense(ego_embeddings, partials, W1, b1.reshape(1, D),
                     W2, b2.reshape(1, D))
